# baseline (speedup 1.0000x reference)
import jax
import jax.numpy as jnp
from jax.experimental import pallas as pl
from jax.experimental.pallas import tpu as pltpu

_N = 8192
_BM = 256
_BN = 2048
_PAIRS_I = (0, 0, 0, 1, 1, 2)
_PAIRS_J = (1, 2, 3, 2, 3, 3)
_NUM_PAIRS = len(_PAIRS_I)


def _bf(x):
    return x.astype(jnp.bfloat16).astype(jnp.float32)


def _transform(x, y, z, P):
    Pb = _bf(P)
    xb, yb, zb = _bf(x), _bf(y), _bf(z)
    wx = Pb[0, 0] * xb + Pb[0, 1] * yb + Pb[0, 2] * zb + Pb[0, 3]
    wy = Pb[1, 0] * xb + Pb[1, 1] * yb + Pb[1, 2] * zb + Pb[1, 3]
    wz = Pb[2, 0] * xb + Pb[2, 1] * yb + Pb[2, 2] * zb + Pb[2, 3]
    return wx, wy, wz


def _chamfer_kernel(a_ref, bT_ref, pose_i_ref, pose_j_ref, out_ref, minsq_ref):
    p = pl.program_id(0)
    m = pl.program_id(1)
    n = pl.program_id(2)
    nb = pl.num_programs(2)

    a = a_ref[0]
    bT = bT_ref[0]

    awx, awy, awz = _transform(a[:, 0:1], a[:, 1:2], a[:, 2:3], pose_i_ref[0])
    bwx, bwy, bwz = _transform(bT[0:1, :], bT[1:2, :], bT[2:3, :], pose_j_ref[0])

    a2 = awx * awx + awy * awy + awz * awz
    b2 = bwx * bwx + bwy * bwy + bwz * bwz

    ab = (_bf(awx) * _bf(bwx)
          + _bf(awy) * _bf(bwy)
          + _bf(awz) * _bf(bwz))
    d2 = a2 + b2 - 2.0 * ab
    bmin = jnp.min(d2, axis=1, keepdims=True)

    @pl.when(n == 0)
    def _init_min():
        minsq_ref[...] = bmin

    @pl.when(n != 0)
    def _update_min():
        minsq_ref[...] = jnp.minimum(minsq_ref[...], bmin)

    @pl.when((p == 0) & (m == 0) & (n == 0))
    def _init_out():
        out_ref[...] = jnp.zeros((1, 1), jnp.float32)

    @pl.when(n == nb - 1)
    def _accumulate():
        dist = jnp.sqrt(jnp.maximum(minsq_ref[...], 1e-12))
        out_ref[...] += jnp.sum(dist).reshape(1, 1)


def kernel(point_clouds, camera_poses):
    iarr = jnp.array(_PAIRS_I, dtype=jnp.int32)
    jarr = jnp.array(_PAIRS_J, dtype=jnp.int32)
    a_pairs = point_clouds[iarr]
    bT_pairs = jnp.transpose(point_clouds[jarr], (0, 2, 1))
    poses_i = camera_poses[iarr]
    poses_j = camera_poses[jarr]

    total = pl.pallas_call(
        _chamfer_kernel,
        grid=(_NUM_PAIRS, _N // _BM, _N // _BN),
        in_specs=[
            pl.BlockSpec((1, _BM, 3), lambda p, m, n: (p, m, 0)),
            pl.BlockSpec((1, 3, _BN), lambda p, m, n: (p, 0, n)),
            pl.BlockSpec((1, 4, 4), lambda p, m, n: (p, 0, 0)),
            pl.BlockSpec((1, 4, 4), lambda p, m, n: (p, 0, 0)),
        ],
        out_specs=pl.BlockSpec((1, 1), lambda p, m, n: (0, 0)),
        out_shape=jax.ShapeDtypeStruct((1, 1), jnp.float32),
        scratch_shapes=[pltpu.VMEM((_BM, 1), jnp.float32)],
    )(a_pairs, bT_pairs, poses_i, poses_j)

    return total[0, 0] / (_NUM_PAIRS * _N)
